# baseline (speedup 1.0000x reference)
import functools

import jax
import jax.numpy as jnp
from jax import lax
from jax.experimental import pallas as pl
from jax.experimental.pallas import tpu as pltpu
from jax.experimental.pallas import tpu_sc as plsc

D = 64
NC = 2
NS = 16
NW = NC * NS
FLD = 26
BB = 128
NBUF = 2


@jax.jit
def _sc_gather(idx_fmaj, table):
    batch = idx_fmaj.shape[0] // FLD
    nbc = batch // BB
    nblk = FLD * nbc
    blk_per_w = nblk // NW
    mesh = plsc.VectorSubcoreMesh(core_axis_name="c", subcore_axis_name="s")

    @functools.partial(
        pl.kernel,
        out_type=jax.ShapeDtypeStruct((FLD, 8, nbc, 8, BB), jnp.float32),
        mesh=mesh,
        scratch_types=[
            pltpu.VMEM((NBUF, BB), jnp.int32),
            pltpu.VMEM((NBUF, BB, D), jnp.float32),
            pltpu.VMEM((NBUF, D, BB), jnp.float32),
            [pltpu.SemaphoreType.DMA] * NBUF,
            [pltpu.SemaphoreType.DMA] * NBUF,
        ],
        compiler_params=pltpu.CompilerParams(
            use_tc_tiling_on_sc=False, needs_layout_passes=False
        ),
    )
    def k(idx_hbm, tab_hbm, out_hbm, idx_v, rows_v, tr_v, gsems, wsems):
        wid = lax.axis_index("s") * NC + lax.axis_index("c")
        blk0 = wid * blk_per_w
        lane = lax.iota(jnp.int32, 16)

        def start_gather(kk, p):
            blk = blk0 + kk
            pltpu.sync_copy(idx_hbm.at[pl.ds(blk * BB, BB)], idx_v.at[p])
            pltpu.async_copy(tab_hbm.at[idx_v.at[p]], rows_v.at[p], gsems[p])

        def wait_gather(p):
            pltpu.make_async_copy(
                tab_hbm.at[idx_v.at[p]], rows_v.at[p], gsems[p]
            ).wait()

        def transpose(p):
            src = rows_v.at[p]
            dst = tr_v.at[p]

            def body(d, carry):
                for g in range(BB // 16):
                    rowv = lane + (16 * g)
                    colv = lax.broadcast_in_dim(d, (16,), ())
                    v = plsc.load_gather(src, [rowv, colv])
                    dst[d, pl.ds(16 * g, 16)] = v
                return carry

            lax.fori_loop(0, D, body, 0, unroll=False)

        def start_write(kk, p):
            blk = blk0 + kk
            f = blk // nbc
            bc = blk % nbc
            for d8 in range(8):
                pltpu.async_copy(
                    tr_v.at[p].at[pl.ds(d8 * 8, 8)],
                    out_hbm.at[f, d8, bc],
                    wsems[p],
                )

        def wait_write(p):
            for d8 in range(8):
                pltpu.make_async_copy(
                    tr_v.at[p].at[pl.ds(d8 * 8, 8)],
                    out_hbm.at[0, 0, 0],
                    wsems[p],
                ).wait()

        for p in range(NBUF):
            start_gather(p, p)

        def body(g, carry):
            for p in range(NBUF):
                kk = g * NBUF + p
                wait_gather(p)

                @pl.when(g > 0)
                def _():
                    wait_write(p)

                transpose(p)
                start_write(kk, p)

                @pl.when(kk + NBUF < blk_per_w)
                def _():
                    start_gather(kk + NBUF, p)

            return carry

        lax.fori_loop(0, blk_per_w // NBUF, body, 0, unroll=False)
        for p in range(NBUF):
            wait_write(p)

    return k(idx_fmaj, table)


def kernel(category_ids, embedding_table):
    batch, fields = category_ids.shape
    idx_fmaj = category_ids.T.reshape(batch * fields).astype(jnp.int32)
    out5 = _sc_gather(idx_fmaj, embedding_table)
    return out5.transpose((2, 4, 0, 1, 3)).reshape(batch, fields, D)
